# baseline (speedup 1.0000x reference)
import functools

import jax
import jax.numpy as jnp
from jax import lax
from jax.experimental import pallas as pl
from jax.experimental.pallas import tpu as pltpu
from jax.experimental.pallas import tpu_sc as plsc

N = 100000
D = 128
K = 64
BLK = 32768
NBLK = 4
NPAD = NBLK * BLK
NW = 32
SEG = NPAD // NW
SEG_BLKS = SEG // K
NEG = float("-inf")


def _scores_body(q_ref, m_ref, o_ref):
    i = pl.program_id(0)
    q = q_ref[...]
    qn = q / jnp.maximum(jnp.sqrt(jnp.sum(q * q)), 1e-12)
    m = m_ref[...]
    ss = jnp.sum(m * m, axis=1, keepdims=True)
    mn = m / jnp.maximum(jnp.sqrt(ss), 1e-12)
    sc = lax.dot_general(qn.astype(jnp.bfloat16), mn.astype(jnp.bfloat16),
                         (((1,), (1,)), ((), ())),
                         preferred_element_type=jnp.float32)
    col = lax.broadcasted_iota(jnp.int32, (1, BLK), 1) + i * BLK
    sc = jnp.where(col < N, sc, NEG)
    o_ref[...] = sc.reshape((BLK,))


def _scores(q2, mem):
    return pl.pallas_call(
        _scores_body,
        grid=(NBLK,),
        in_specs=[
            pl.BlockSpec((1, D), lambda i: (0, 0)),
            pl.BlockSpec((BLK, D), lambda i: (i, 0)),
        ],
        out_specs=pl.BlockSpec((BLK,), lambda i: (i,)),
        out_shape=jax.ShapeDtypeStruct((NPAD,), jnp.float32),
    )(q2, mem)


def _rev(x):
    return lax.rev(x, (0,))


def _srt(k, v):
    return plsc.sort_key_val(k, v)


def _cmpx(ka, va, kb, vb):
    m = ka <= kb
    return (jnp.where(m, ka, kb), jnp.where(m, va, vb),
            jnp.where(m, kb, ka), jnp.where(m, vb, va))


def _merge16(ak, av, bk, bv):
    bk, bv = _rev(bk), _rev(bv)
    lok, lov, hik, hiv = _cmpx(ak, av, bk, bv)
    lok, lov = _srt(lok, lov)
    hik, hiv = _srt(hik, hiv)
    return lok, lov, hik, hiv


def _bitonic64(ks, vs):
    k0, k1, k2, k3 = ks
    v0, v1, v2, v3 = vs
    k0, v0, k2, v2 = _cmpx(k0, v0, k2, v2)
    k1, v1, k3, v3 = _cmpx(k1, v1, k3, v3)
    k0, v0, k1, v1 = _cmpx(k0, v0, k1, v1)
    k2, v2, k3, v3 = _cmpx(k2, v2, k3, v3)
    k0, v0 = _srt(k0, v0)
    k1, v1 = _srt(k1, v1)
    k2, v2 = _srt(k2, v2)
    k3, v3 = _srt(k3, v3)
    return [k0, k1, k2, k3], [v0, v1, v2, v3]


def _sort64(ks, vs):
    k0, v0 = _srt(ks[0], vs[0])
    k1, v1 = _srt(ks[1], vs[1])
    k2, v2 = _srt(ks[2], vs[2])
    k3, v3 = _srt(ks[3], vs[3])
    k0, v0, k1, v1 = _merge16(k0, v0, k1, v1)
    k2, v2, k3, v3 = _merge16(k2, v2, k3, v3)
    return _bitonic64([k0, k1, _rev(k3), _rev(k2)],
                      [v0, v1, _rev(v3), _rev(v2)])


def _topk_merge(rk, rv, bk, bv):
    dk = [_rev(bk[3]), _rev(bk[2]), _rev(bk[1]), _rev(bk[0])]
    dv = [_rev(bv[3]), _rev(bv[2]), _rev(bv[1]), _rev(bv[0])]
    ck, cv = [], []
    for c in range(4):
        m = rk[c] >= dk[c]
        ck.append(jnp.where(m, rk[c], dk[c]))
        cv.append(jnp.where(m, rv[c], dv[c]))
    return _bitonic64(ck, cv)


def _tournament(src_k, src_v, dst_k, dst_v, nblk):
    n = nblk
    while n > 1:
        half = n // 2

        @plsc.parallel_loop(0, half, 1, unroll=2)
        def _m(i, _sk=src_k, _sv=src_v, _dk=dst_k, _dv=dst_v):
            o0 = (2 * i) * K
            o1 = (2 * i + 1) * K
            od = i * K
            ak = [_sk[pl.ds(o0 + 16 * c, 16)] for c in range(4)]
            av = [_sv[pl.ds(o0 + 16 * c, 16)] for c in range(4)]
            bk = [_sk[pl.ds(o1 + 16 * c, 16)] for c in range(4)]
            bv = [_sv[pl.ds(o1 + 16 * c, 16)] for c in range(4)]
            nk, nv = _topk_merge(ak, av, bk, bv)
            for c in range(4):
                _dk[pl.ds(od + 16 * c, 16)] = nk[c]
                _dv[pl.ds(od + 16 * c, 16)] = nv[c]

        if n % 2 == 1:
            ol = (n - 1) * K
            od = half * K
            for c in range(4):
                dst_k[pl.ds(od + 16 * c, 16)] = src_k[pl.ds(ol + 16 * c, 16)]
                dst_v[pl.ds(od + 16 * c, 16)] = src_v[pl.ds(ol + 16 * c, 16)]
        n = half + (n % 2)
        src_k, src_v, dst_k, dst_v = dst_k, dst_v, src_k, src_v
    return src_k, src_v


def _partial_body(scores_hbm, ok_hbm, ov_hbm, seg_v, ka_v, va_v, kb_v, vb_v):
    cid = lax.axis_index("c")
    sid = lax.axis_index("s")
    wid = sid * 2 + cid
    base = wid * SEG
    pltpu.sync_copy(scores_hbm.at[pl.ds(base, SEG)], seg_v)
    iota = lax.iota(jnp.int32, 16)

    @plsc.parallel_loop(0, SEG_BLKS, 1, unroll=2)
    def _p1(j):
        off = j * K
        ks = [seg_v[pl.ds(off + 16 * c, 16)] for c in range(4)]
        vs = [iota + (base + off + 16 * c) for c in range(4)]
        sk, sv = _sort64(ks, vs)
        for c in range(4):
            ka_v[pl.ds(off + 16 * c, 16)] = sk[c]
            va_v[pl.ds(off + 16 * c, 16)] = sv[c]

    rk, rv = _tournament(ka_v, va_v, kb_v, vb_v, SEG_BLKS)
    pltpu.sync_copy(rk.at[pl.ds(0, K)], ok_hbm.at[pl.ds(wid * K, K)])
    pltpu.sync_copy(rv.at[pl.ds(0, K)], ov_hbm.at[pl.ds(wid * K, K)])


def _partial_topk(scores):
    mesh = plsc.VectorSubcoreMesh(core_axis_name="c", subcore_axis_name="s",
                                  num_cores=2, num_subcores=16)
    f = functools.partial(
        pl.kernel,
        out_type=[jax.ShapeDtypeStruct((NW * K,), jnp.float32),
                  jax.ShapeDtypeStruct((NW * K,), jnp.int32)],
        mesh=mesh,
        compiler_params=pltpu.CompilerParams(needs_layout_passes=False),
        scratch_types=[pltpu.VMEM((SEG,), jnp.float32),
                       pltpu.VMEM((SEG,), jnp.float32),
                       pltpu.VMEM((SEG,), jnp.int32),
                       pltpu.VMEM((SEG,), jnp.float32),
                       pltpu.VMEM((SEG,), jnp.int32)],
    )(_partial_body)
    return f(scores)


def _final_body(pk_hbm, pv_hbm, mem_hbm, sh_hbm, ret_hbm, ts_hbm,
                pk_v, pv_v, kb_v, vb_v, sh_v, idx_v, rows_v, ks_v, sem):
    cid = lax.axis_index("c")
    sid = lax.axis_index("s")
    wid = sid * 2 + cid

    @pl.when(wid == 0)
    def _():
        pltpu.sync_copy(pk_hbm, pk_v)
        pltpu.sync_copy(pv_hbm, pv_v)
        pltpu.sync_copy(sh_hbm, sh_v)

        rk, rv = _tournament(pk_v, pv_v, kb_v, vb_v, NW)

        sh = sh_v[...]
        for c in range(4):
            ks_v[pl.ds(16 * c, 16)] = _rev(rk[pl.ds(16 * (3 - c), 16)])
            iv = _rev(rv[pl.ds(16 * (3 - c), 16)]) + sh
            iv = jnp.minimum(jnp.maximum(iv, 0), N - 1)
            idx_v[pl.ds(16 * c, 16)] = iv
        pltpu.async_copy(mem_hbm.at[idx_v], rows_v, sem).wait()
        pltpu.sync_copy(rows_v, ret_hbm)
        pltpu.sync_copy(ks_v, ts_hbm)


def _final(pk, pv, mem, shift):
    mesh = plsc.VectorSubcoreMesh(core_axis_name="c", subcore_axis_name="s",
                                  num_cores=2, num_subcores=16)
    f = functools.partial(
        pl.kernel,
        out_type=[jax.ShapeDtypeStruct((K, D), jnp.float32),
                  jax.ShapeDtypeStruct((K,), jnp.float32)],
        mesh=mesh,
        compiler_params=pltpu.CompilerParams(needs_layout_passes=False),
        scratch_types=[pltpu.VMEM((NW * K,), jnp.float32),
                       pltpu.VMEM((NW * K,), jnp.int32),
                       pltpu.VMEM((NW * K,), jnp.float32),
                       pltpu.VMEM((NW * K,), jnp.int32),
                       pltpu.VMEM((16,), jnp.int32),
                       pltpu.VMEM((K,), jnp.int32),
                       pltpu.VMEM((K, D), jnp.float32),
                       pltpu.VMEM((K,), jnp.float32),
                       pltpu.SemaphoreType.DMA],
    )(_final_body)
    return f(pk, pv, mem, shift)


NPAD2 = 100352
NSEG = 16
SEG2 = NPAD2 // NSEG
SEG2_BLKS = SEG2 // K


def _fused_body(scores_hbm, mem_hbm, sh_hbm, ret_hbm, ts_hbm,
                seg_v, ka_v, va_v, kb_v, vb_v, sh_v, idx_v, rows_v, ks_v,
                shared_k, shared_v, sem):
    sid = lax.axis_index("s")
    base = sid * SEG2
    pltpu.sync_copy(scores_hbm.at[pl.ds(base, SEG2)], seg_v)
    iota = lax.iota(jnp.int32, 16)

    @plsc.parallel_loop(0, SEG2_BLKS, 1, unroll=2)
    def _p1(j):
        off = j * K
        ks = [seg_v[pl.ds(off + 16 * c, 16)] for c in range(4)]
        vs = [iota + (base + off + 16 * c) for c in range(4)]
        sk, sv = _sort64(ks, vs)
        for c in range(4):
            ka_v[pl.ds(off + 16 * c, 16)] = sk[c]
            va_v[pl.ds(off + 16 * c, 16)] = sv[c]

    rk, rv = _tournament(ka_v, va_v, kb_v, vb_v, SEG2_BLKS)
    pltpu.sync_copy(rk.at[pl.ds(0, K)], shared_k.at[pl.ds(sid * K, K)])
    pltpu.sync_copy(rv.at[pl.ds(0, K)], shared_v.at[pl.ds(sid * K, K)])
    plsc.subcore_barrier()

    @pl.when(sid == 0)
    def _():
        pltpu.sync_copy(shared_k, ka_v.at[pl.ds(0, NSEG * K)])
        pltpu.sync_copy(shared_v, va_v.at[pl.ds(0, NSEG * K)])
        pltpu.sync_copy(sh_hbm, sh_v)
        fk, fv = _tournament(ka_v, va_v, kb_v, vb_v, NSEG)
        sh = sh_v[...]
        for c in range(4):
            ks_v[pl.ds(16 * c, 16)] = _rev(fk[pl.ds(16 * (3 - c), 16)])
            iv = _rev(fv[pl.ds(16 * (3 - c), 16)]) + sh
            iv = jnp.minimum(jnp.maximum(iv, 0), N - 1)
            idx_v[pl.ds(16 * c, 16)] = iv
        pltpu.async_copy(mem_hbm.at[idx_v], rows_v, sem).wait()
        pltpu.sync_copy(rows_v, ret_hbm)
        pltpu.sync_copy(ks_v, ts_hbm)


def _fused_topk(scores, mem, shift):
    mesh = plsc.VectorSubcoreMesh(core_axis_name="c", subcore_axis_name="s",
                                  num_cores=2, num_subcores=16)
    f = functools.partial(
        pl.kernel,
        out_type=[jax.ShapeDtypeStruct((K, D), jnp.float32),
                  jax.ShapeDtypeStruct((K,), jnp.float32)],
        mesh=mesh,
        compiler_params=pltpu.CompilerParams(needs_layout_passes=False),
        scratch_types=[pltpu.VMEM((SEG2,), jnp.float32),
                       pltpu.VMEM((SEG2,), jnp.float32),
                       pltpu.VMEM((SEG2,), jnp.int32),
                       pltpu.VMEM((SEG2,), jnp.float32),
                       pltpu.VMEM((SEG2,), jnp.int32),
                       pltpu.VMEM((16,), jnp.int32),
                       pltpu.VMEM((K,), jnp.int32),
                       pltpu.VMEM((K, D), jnp.float32),
                       pltpu.VMEM((K,), jnp.float32),
                       pltpu.VMEM_SHARED((NSEG * K,), jnp.float32),
                       pltpu.VMEM_SHARED((NSEG * K,), jnp.int32),
                       pltpu.SemaphoreType.DMA],
    )(_fused_body)
    return f(scores, mem, shift)


def kernel(query, memory_features, k):
    q2 = query.reshape(1, D).astype(jnp.float32)
    scores = _scores(q2, memory_features)
    shift = jnp.broadcast_to(jnp.asarray(k, jnp.int32) - K, (16,))
    retrieved, top_scores = _fused_topk(scores, memory_features, shift)
    return retrieved, top_scores


# speedup vs baseline: 1.1402x; 1.1402x over previous
import functools

import jax
import jax.numpy as jnp
from jax import lax
from jax.experimental import pallas as pl
from jax.experimental.pallas import tpu as pltpu
from jax.experimental.pallas import tpu_sc as plsc

N = 100000
D = 128
K = 64
BLK = 20480
NBLK = 5
NPAD = NBLK * BLK
NW = 32
SEG = NPAD // NW
SEG_BLKS = SEG // K
NEG = float("-inf")


def _scores_body(q_ref, m_ref, o_ref):
    i = pl.program_id(0)
    q = q_ref[...]
    qn = q / jnp.maximum(jnp.sqrt(jnp.sum(q * q)), 1e-12)
    m = m_ref[...]
    ss = jnp.sum(m * m, axis=1, keepdims=True)
    mn = m / jnp.maximum(jnp.sqrt(ss), 1e-12)
    sc = lax.dot_general(qn.astype(jnp.bfloat16), mn.astype(jnp.bfloat16),
                         (((1,), (1,)), ((), ())),
                         preferred_element_type=jnp.float32)
    col = lax.broadcasted_iota(jnp.int32, (1, BLK), 1) + i * BLK
    sc = jnp.where(col < N, sc, NEG)
    o_ref[...] = sc.reshape((BLK,))


def _scores(q2, mem):
    return pl.pallas_call(
        _scores_body,
        grid=(NBLK,),
        in_specs=[
            pl.BlockSpec((1, D), lambda i: (0, 0)),
            pl.BlockSpec((BLK, D), lambda i: (i, 0)),
        ],
        out_specs=pl.BlockSpec((BLK,), lambda i: (i,)),
        out_shape=jax.ShapeDtypeStruct((NPAD,), jnp.float32),
    )(q2, mem)


def _rev(x):
    return lax.rev(x, (0,))


def _srt(k, v):
    return plsc.sort_key_val(k, v)


def _cmpx(ka, va, kb, vb):
    m = ka <= kb
    return (jnp.where(m, ka, kb), jnp.where(m, va, vb),
            jnp.where(m, kb, ka), jnp.where(m, vb, va))


def _merge16(ak, av, bk, bv):
    bk, bv = _rev(bk), _rev(bv)
    lok, lov, hik, hiv = _cmpx(ak, av, bk, bv)
    lok, lov = _srt(lok, lov)
    hik, hiv = _srt(hik, hiv)
    return lok, lov, hik, hiv


def _bitonic64(ks, vs):
    k0, k1, k2, k3 = ks
    v0, v1, v2, v3 = vs
    k0, v0, k2, v2 = _cmpx(k0, v0, k2, v2)
    k1, v1, k3, v3 = _cmpx(k1, v1, k3, v3)
    k0, v0, k1, v1 = _cmpx(k0, v0, k1, v1)
    k2, v2, k3, v3 = _cmpx(k2, v2, k3, v3)
    k0, v0 = _srt(k0, v0)
    k1, v1 = _srt(k1, v1)
    k2, v2 = _srt(k2, v2)
    k3, v3 = _srt(k3, v3)
    return [k0, k1, k2, k3], [v0, v1, v2, v3]


def _sort64(ks, vs):
    k0, v0 = _srt(ks[0], vs[0])
    k1, v1 = _srt(ks[1], vs[1])
    k2, v2 = _srt(ks[2], vs[2])
    k3, v3 = _srt(ks[3], vs[3])
    k0, v0, k1, v1 = _merge16(k0, v0, k1, v1)
    k2, v2, k3, v3 = _merge16(k2, v2, k3, v3)
    return _bitonic64([k0, k1, _rev(k3), _rev(k2)],
                      [v0, v1, _rev(v3), _rev(v2)])


def _topk_merge(rk, rv, bk, bv):
    dk = [_rev(bk[3]), _rev(bk[2]), _rev(bk[1]), _rev(bk[0])]
    dv = [_rev(bv[3]), _rev(bv[2]), _rev(bv[1]), _rev(bv[0])]
    ck, cv = [], []
    for c in range(4):
        m = rk[c] >= dk[c]
        ck.append(jnp.where(m, rk[c], dk[c]))
        cv.append(jnp.where(m, rv[c], dv[c]))
    return _bitonic64(ck, cv)


def _tournament(src_k, src_v, dst_k, dst_v, nblk):
    n = nblk
    while n > 1:
        half = n // 2

        @plsc.parallel_loop(0, half, 1, unroll=2)
        def _m(i, _sk=src_k, _sv=src_v, _dk=dst_k, _dv=dst_v):
            o0 = (2 * i) * K
            o1 = (2 * i + 1) * K
            od = i * K
            ak = [_sk[pl.ds(o0 + 16 * c, 16)] for c in range(4)]
            av = [_sv[pl.ds(o0 + 16 * c, 16)] for c in range(4)]
            bk = [_sk[pl.ds(o1 + 16 * c, 16)] for c in range(4)]
            bv = [_sv[pl.ds(o1 + 16 * c, 16)] for c in range(4)]
            nk, nv = _topk_merge(ak, av, bk, bv)
            for c in range(4):
                _dk[pl.ds(od + 16 * c, 16)] = nk[c]
                _dv[pl.ds(od + 16 * c, 16)] = nv[c]

        if n % 2 == 1:
            ol = (n - 1) * K
            od = half * K
            for c in range(4):
                dst_k[pl.ds(od + 16 * c, 16)] = src_k[pl.ds(ol + 16 * c, 16)]
                dst_v[pl.ds(od + 16 * c, 16)] = src_v[pl.ds(ol + 16 * c, 16)]
        n = half + (n % 2)
        src_k, src_v, dst_k, dst_v = dst_k, dst_v, src_k, src_v
    return src_k, src_v


def _partial_body(scores_hbm, ok_hbm, ov_hbm, seg_v, ka_v, va_v, kb_v, vb_v):
    cid = lax.axis_index("c")
    sid = lax.axis_index("s")
    wid = sid * 2 + cid
    base = wid * SEG
    pltpu.sync_copy(scores_hbm.at[pl.ds(base, SEG)], seg_v)
    iota = lax.iota(jnp.int32, 16)

    @plsc.parallel_loop(0, SEG_BLKS, 1, unroll=2)
    def _p1(j):
        off = j * K
        ks = [seg_v[pl.ds(off + 16 * c, 16)] for c in range(4)]
        vs = [iota + (base + off + 16 * c) for c in range(4)]
        sk, sv = _sort64(ks, vs)
        for c in range(4):
            ka_v[pl.ds(off + 16 * c, 16)] = sk[c]
            va_v[pl.ds(off + 16 * c, 16)] = sv[c]

    rk, rv = _tournament(ka_v, va_v, kb_v, vb_v, SEG_BLKS)
    pltpu.sync_copy(rk.at[pl.ds(0, K)], ok_hbm.at[pl.ds(wid * K, K)])
    pltpu.sync_copy(rv.at[pl.ds(0, K)], ov_hbm.at[pl.ds(wid * K, K)])


def _partial_topk(scores):
    mesh = plsc.VectorSubcoreMesh(core_axis_name="c", subcore_axis_name="s",
                                  num_cores=2, num_subcores=16)
    f = functools.partial(
        pl.kernel,
        out_type=[jax.ShapeDtypeStruct((NW * K,), jnp.float32),
                  jax.ShapeDtypeStruct((NW * K,), jnp.int32)],
        mesh=mesh,
        compiler_params=pltpu.CompilerParams(needs_layout_passes=False),
        scratch_types=[pltpu.VMEM((SEG,), jnp.float32),
                       pltpu.VMEM((SEG,), jnp.float32),
                       pltpu.VMEM((SEG,), jnp.int32),
                       pltpu.VMEM((SEG,), jnp.float32),
                       pltpu.VMEM((SEG,), jnp.int32)],
    )(_partial_body)
    return f(scores)


def _final_body(pk_hbm, pv_hbm, mem_hbm, sh_hbm, ret_hbm, ts_hbm,
                pk_v, pv_v, kb_v, vb_v, sh_v, idx_v, rows_v, ks_v, sem):
    cid = lax.axis_index("c")
    sid = lax.axis_index("s")
    wid = sid * 2 + cid

    @pl.when(wid == 0)
    def _():
        pltpu.sync_copy(pk_hbm, pk_v)
        pltpu.sync_copy(pv_hbm, pv_v)
        pltpu.sync_copy(sh_hbm, sh_v)

        rk, rv = _tournament(pk_v, pv_v, kb_v, vb_v, NW)

        sh = sh_v[...]
        for c in range(4):
            ks_v[pl.ds(16 * c, 16)] = _rev(rk[pl.ds(16 * (3 - c), 16)])
            iv = _rev(rv[pl.ds(16 * (3 - c), 16)]) + sh
            iv = jnp.minimum(jnp.maximum(iv, 0), N - 1)
            idx_v[pl.ds(16 * c, 16)] = iv
        pltpu.async_copy(mem_hbm.at[idx_v], rows_v, sem).wait()
        pltpu.sync_copy(rows_v, ret_hbm)
        pltpu.sync_copy(ks_v, ts_hbm)


def _final(pk, pv, mem, shift):
    mesh = plsc.VectorSubcoreMesh(core_axis_name="c", subcore_axis_name="s",
                                  num_cores=2, num_subcores=16)
    f = functools.partial(
        pl.kernel,
        out_type=[jax.ShapeDtypeStruct((K, D), jnp.float32),
                  jax.ShapeDtypeStruct((K,), jnp.float32)],
        mesh=mesh,
        compiler_params=pltpu.CompilerParams(needs_layout_passes=False),
        scratch_types=[pltpu.VMEM((NW * K,), jnp.float32),
                       pltpu.VMEM((NW * K,), jnp.int32),
                       pltpu.VMEM((NW * K,), jnp.float32),
                       pltpu.VMEM((NW * K,), jnp.int32),
                       pltpu.VMEM((16,), jnp.int32),
                       pltpu.VMEM((K,), jnp.int32),
                       pltpu.VMEM((K, D), jnp.float32),
                       pltpu.VMEM((K,), jnp.float32),
                       pltpu.SemaphoreType.DMA],
    )(_final_body)
    return f(pk, pv, mem, shift)


NPAD2 = 100352
NSEG = 16
SEG2 = NPAD2 // NSEG
SEG2_BLKS = SEG2 // K


def _fused_body(scores_hbm, mem_hbm, sh_hbm, ret_hbm, ts_hbm,
                seg_v, ka_v, va_v, kb_v, vb_v, sh_v, idx_v, rows_v, ks_v,
                shared_k, shared_v, sem):
    sid = lax.axis_index("s")
    base = sid * SEG2
    pltpu.sync_copy(scores_hbm.at[pl.ds(base, SEG2)], seg_v)
    iota = lax.iota(jnp.int32, 16)

    @plsc.parallel_loop(0, SEG2_BLKS, 1, unroll=2)
    def _p1(j):
        off = j * K
        ks = [seg_v[pl.ds(off + 16 * c, 16)] for c in range(4)]
        vs = [iota + (base + off + 16 * c) for c in range(4)]
        sk, sv = _sort64(ks, vs)
        for c in range(4):
            ka_v[pl.ds(off + 16 * c, 16)] = sk[c]
            va_v[pl.ds(off + 16 * c, 16)] = sv[c]

    rk, rv = _tournament(ka_v, va_v, kb_v, vb_v, SEG2_BLKS)
    pltpu.sync_copy(rk.at[pl.ds(0, K)], shared_k.at[pl.ds(sid * K, K)])
    pltpu.sync_copy(rv.at[pl.ds(0, K)], shared_v.at[pl.ds(sid * K, K)])
    plsc.subcore_barrier()

    @pl.when(sid == 0)
    def _():
        pltpu.sync_copy(shared_k, ka_v.at[pl.ds(0, NSEG * K)])
        pltpu.sync_copy(shared_v, va_v.at[pl.ds(0, NSEG * K)])
        pltpu.sync_copy(sh_hbm, sh_v)
        fk, fv = _tournament(ka_v, va_v, kb_v, vb_v, NSEG)
        sh = sh_v[...]
        for c in range(4):
            ks_v[pl.ds(16 * c, 16)] = _rev(fk[pl.ds(16 * (3 - c), 16)])
            iv = _rev(fv[pl.ds(16 * (3 - c), 16)]) + sh
            iv = jnp.minimum(jnp.maximum(iv, 0), N - 1)
            idx_v[pl.ds(16 * c, 16)] = iv
        pltpu.async_copy(mem_hbm.at[idx_v], rows_v, sem).wait()
        pltpu.sync_copy(rows_v, ret_hbm)
        pltpu.sync_copy(ks_v, ts_hbm)


def _fused_topk(scores, mem, shift):
    mesh = plsc.VectorSubcoreMesh(core_axis_name="c", subcore_axis_name="s",
                                  num_cores=2, num_subcores=16)
    f = functools.partial(
        pl.kernel,
        out_type=[jax.ShapeDtypeStruct((K, D), jnp.float32),
                  jax.ShapeDtypeStruct((K,), jnp.float32)],
        mesh=mesh,
        compiler_params=pltpu.CompilerParams(needs_layout_passes=False),
        scratch_types=[pltpu.VMEM((SEG2,), jnp.float32),
                       pltpu.VMEM((SEG2,), jnp.float32),
                       pltpu.VMEM((SEG2,), jnp.int32),
                       pltpu.VMEM((SEG2,), jnp.float32),
                       pltpu.VMEM((SEG2,), jnp.int32),
                       pltpu.VMEM((16,), jnp.int32),
                       pltpu.VMEM((K,), jnp.int32),
                       pltpu.VMEM((K, D), jnp.float32),
                       pltpu.VMEM((K,), jnp.float32),
                       pltpu.VMEM_SHARED((NSEG * K,), jnp.float32),
                       pltpu.VMEM_SHARED((NSEG * K,), jnp.int32),
                       pltpu.SemaphoreType.DMA],
    )(_fused_body)
    return f(scores, mem, shift)


def kernel(query, memory_features, k):
    q2 = query.reshape(1, D).astype(jnp.float32)
    scores = _scores(q2, memory_features)
    shift = jnp.broadcast_to(jnp.asarray(k, jnp.int32) - K, (16,))
    retrieved, top_scores = _fused_topk(scores, memory_features, shift)
    return retrieved, top_scores


# speedup vs baseline: 1.1535x; 1.0117x over previous
import functools

import jax
import jax.numpy as jnp
from jax import lax
from jax.experimental import pallas as pl
from jax.experimental.pallas import tpu as pltpu
from jax.experimental.pallas import tpu_sc as plsc

N = 100000
D = 128
K = 64
BLK = 14336
NBLK = 7
NPAD = NBLK * BLK
NW = 32
SEG = NPAD // NW
SEG_BLKS = SEG // K
NEG = float("-inf")


def _scores_body(q_ref, m_ref, o_ref):
    i = pl.program_id(0)
    q = q_ref[...]
    qn = q / jnp.maximum(jnp.sqrt(jnp.sum(q * q)), 1e-12)
    m = m_ref[...]
    ss = jnp.sum(m * m, axis=1, keepdims=True)
    mn = m / jnp.maximum(jnp.sqrt(ss), 1e-12)
    sc = lax.dot_general(qn.astype(jnp.bfloat16), mn.astype(jnp.bfloat16),
                         (((1,), (1,)), ((), ())),
                         preferred_element_type=jnp.float32)
    col = lax.broadcasted_iota(jnp.int32, (1, BLK), 1) + i * BLK
    sc = jnp.where(col < N, sc, NEG)
    o_ref[...] = sc.reshape((BLK,))


def _scores(q2, mem):
    return pl.pallas_call(
        _scores_body,
        grid=(NBLK,),
        in_specs=[
            pl.BlockSpec((1, D), lambda i: (0, 0)),
            pl.BlockSpec((BLK, D), lambda i: (i, 0)),
        ],
        out_specs=pl.BlockSpec((BLK,), lambda i: (i,)),
        out_shape=jax.ShapeDtypeStruct((NPAD,), jnp.float32),
    )(q2, mem)


def _rev(x):
    return lax.rev(x, (0,))


def _srt(k, v):
    return plsc.sort_key_val(k, v)


def _cmpx(ka, va, kb, vb):
    m = ka <= kb
    return (jnp.where(m, ka, kb), jnp.where(m, va, vb),
            jnp.where(m, kb, ka), jnp.where(m, vb, va))


def _merge16(ak, av, bk, bv):
    bk, bv = _rev(bk), _rev(bv)
    lok, lov, hik, hiv = _cmpx(ak, av, bk, bv)
    lok, lov = _srt(lok, lov)
    hik, hiv = _srt(hik, hiv)
    return lok, lov, hik, hiv


def _bitonic64(ks, vs):
    k0, k1, k2, k3 = ks
    v0, v1, v2, v3 = vs
    k0, v0, k2, v2 = _cmpx(k0, v0, k2, v2)
    k1, v1, k3, v3 = _cmpx(k1, v1, k3, v3)
    k0, v0, k1, v1 = _cmpx(k0, v0, k1, v1)
    k2, v2, k3, v3 = _cmpx(k2, v2, k3, v3)
    k0, v0 = _srt(k0, v0)
    k1, v1 = _srt(k1, v1)
    k2, v2 = _srt(k2, v2)
    k3, v3 = _srt(k3, v3)
    return [k0, k1, k2, k3], [v0, v1, v2, v3]


def _sort64(ks, vs):
    k0, v0 = _srt(ks[0], vs[0])
    k1, v1 = _srt(ks[1], vs[1])
    k2, v2 = _srt(ks[2], vs[2])
    k3, v3 = _srt(ks[3], vs[3])
    k0, v0, k1, v1 = _merge16(k0, v0, k1, v1)
    k2, v2, k3, v3 = _merge16(k2, v2, k3, v3)
    return _bitonic64([k0, k1, _rev(k3), _rev(k2)],
                      [v0, v1, _rev(v3), _rev(v2)])


def _topk_merge(rk, rv, bk, bv):
    dk = [_rev(bk[3]), _rev(bk[2]), _rev(bk[1]), _rev(bk[0])]
    dv = [_rev(bv[3]), _rev(bv[2]), _rev(bv[1]), _rev(bv[0])]
    ck, cv = [], []
    for c in range(4):
        m = rk[c] >= dk[c]
        ck.append(jnp.where(m, rk[c], dk[c]))
        cv.append(jnp.where(m, rv[c], dv[c]))
    return _bitonic64(ck, cv)


def _tournament(src_k, src_v, dst_k, dst_v, nblk):
    n = nblk
    while n > 1:
        half = n // 2

        @plsc.parallel_loop(0, half, 1, unroll=2)
        def _m(i, _sk=src_k, _sv=src_v, _dk=dst_k, _dv=dst_v):
            o0 = (2 * i) * K
            o1 = (2 * i + 1) * K
            od = i * K
            ak = [_sk[pl.ds(o0 + 16 * c, 16)] for c in range(4)]
            av = [_sv[pl.ds(o0 + 16 * c, 16)] for c in range(4)]
            bk = [_sk[pl.ds(o1 + 16 * c, 16)] for c in range(4)]
            bv = [_sv[pl.ds(o1 + 16 * c, 16)] for c in range(4)]
            nk, nv = _topk_merge(ak, av, bk, bv)
            for c in range(4):
                _dk[pl.ds(od + 16 * c, 16)] = nk[c]
                _dv[pl.ds(od + 16 * c, 16)] = nv[c]

        if n % 2 == 1:
            ol = (n - 1) * K
            od = half * K
            for c in range(4):
                dst_k[pl.ds(od + 16 * c, 16)] = src_k[pl.ds(ol + 16 * c, 16)]
                dst_v[pl.ds(od + 16 * c, 16)] = src_v[pl.ds(ol + 16 * c, 16)]
        n = half + (n % 2)
        src_k, src_v, dst_k, dst_v = dst_k, dst_v, src_k, src_v
    return src_k, src_v


def _partial_body(scores_hbm, ok_hbm, ov_hbm, seg_v, ka_v, va_v, kb_v, vb_v):
    cid = lax.axis_index("c")
    sid = lax.axis_index("s")
    wid = sid * 2 + cid
    base = wid * SEG
    pltpu.sync_copy(scores_hbm.at[pl.ds(base, SEG)], seg_v)
    iota = lax.iota(jnp.int32, 16)

    @plsc.parallel_loop(0, SEG_BLKS, 1, unroll=2)
    def _p1(j):
        off = j * K
        ks = [seg_v[pl.ds(off + 16 * c, 16)] for c in range(4)]
        vs = [iota + (base + off + 16 * c) for c in range(4)]
        sk, sv = _sort64(ks, vs)
        for c in range(4):
            ka_v[pl.ds(off + 16 * c, 16)] = sk[c]
            va_v[pl.ds(off + 16 * c, 16)] = sv[c]

    rk, rv = _tournament(ka_v, va_v, kb_v, vb_v, SEG_BLKS)
    pltpu.sync_copy(rk.at[pl.ds(0, K)], ok_hbm.at[pl.ds(wid * K, K)])
    pltpu.sync_copy(rv.at[pl.ds(0, K)], ov_hbm.at[pl.ds(wid * K, K)])


def _partial_topk(scores):
    mesh = plsc.VectorSubcoreMesh(core_axis_name="c", subcore_axis_name="s",
                                  num_cores=2, num_subcores=16)
    f = functools.partial(
        pl.kernel,
        out_type=[jax.ShapeDtypeStruct((NW * K,), jnp.float32),
                  jax.ShapeDtypeStruct((NW * K,), jnp.int32)],
        mesh=mesh,
        compiler_params=pltpu.CompilerParams(needs_layout_passes=False),
        scratch_types=[pltpu.VMEM((SEG,), jnp.float32),
                       pltpu.VMEM((SEG,), jnp.float32),
                       pltpu.VMEM((SEG,), jnp.int32),
                       pltpu.VMEM((SEG,), jnp.float32),
                       pltpu.VMEM((SEG,), jnp.int32)],
    )(_partial_body)
    return f(scores)


def _final_body(pk_hbm, pv_hbm, mem_hbm, sh_hbm, ret_hbm, ts_hbm,
                pk_v, pv_v, kb_v, vb_v, sh_v, idx_v, rows_v, ks_v, sem):
    cid = lax.axis_index("c")
    sid = lax.axis_index("s")
    wid = sid * 2 + cid

    @pl.when(wid == 0)
    def _():
        pltpu.sync_copy(pk_hbm, pk_v)
        pltpu.sync_copy(pv_hbm, pv_v)
        pltpu.sync_copy(sh_hbm, sh_v)

        rk, rv = _tournament(pk_v, pv_v, kb_v, vb_v, NW)

        sh = sh_v[...]
        for c in range(4):
            ks_v[pl.ds(16 * c, 16)] = _rev(rk[pl.ds(16 * (3 - c), 16)])
            iv = _rev(rv[pl.ds(16 * (3 - c), 16)]) + sh
            iv = jnp.minimum(jnp.maximum(iv, 0), N - 1)
            idx_v[pl.ds(16 * c, 16)] = iv
        pltpu.async_copy(mem_hbm.at[idx_v], rows_v, sem).wait()
        pltpu.sync_copy(rows_v, ret_hbm)
        pltpu.sync_copy(ks_v, ts_hbm)


def _final(pk, pv, mem, shift):
    mesh = plsc.VectorSubcoreMesh(core_axis_name="c", subcore_axis_name="s",
                                  num_cores=2, num_subcores=16)
    f = functools.partial(
        pl.kernel,
        out_type=[jax.ShapeDtypeStruct((K, D), jnp.float32),
                  jax.ShapeDtypeStruct((K,), jnp.float32)],
        mesh=mesh,
        compiler_params=pltpu.CompilerParams(needs_layout_passes=False),
        scratch_types=[pltpu.VMEM((NW * K,), jnp.float32),
                       pltpu.VMEM((NW * K,), jnp.int32),
                       pltpu.VMEM((NW * K,), jnp.float32),
                       pltpu.VMEM((NW * K,), jnp.int32),
                       pltpu.VMEM((16,), jnp.int32),
                       pltpu.VMEM((K,), jnp.int32),
                       pltpu.VMEM((K, D), jnp.float32),
                       pltpu.VMEM((K,), jnp.float32),
                       pltpu.SemaphoreType.DMA],
    )(_final_body)
    return f(pk, pv, mem, shift)


NPAD2 = 100352
NSEG = 16
SEG2 = NPAD2 // NSEG
SEG2_BLKS = SEG2 // K


def _fused_body(scores_hbm, mem_hbm, sh_hbm, ret_hbm, ts_hbm,
                seg_v, ka_v, va_v, kb_v, vb_v, sh_v, idx_v, rows_v, ks_v,
                shared_k, shared_v, sem):
    sid = lax.axis_index("s")
    base = sid * SEG2
    pltpu.sync_copy(scores_hbm.at[pl.ds(base, SEG2)], seg_v)
    iota = lax.iota(jnp.int32, 16)

    @plsc.parallel_loop(0, SEG2_BLKS, 1, unroll=2)
    def _p1(j):
        off = j * K
        ks = [seg_v[pl.ds(off + 16 * c, 16)] for c in range(4)]
        vs = [iota + (base + off + 16 * c) for c in range(4)]
        sk, sv = _sort64(ks, vs)
        for c in range(4):
            ka_v[pl.ds(off + 16 * c, 16)] = sk[c]
            va_v[pl.ds(off + 16 * c, 16)] = sv[c]

    rk, rv = _tournament(ka_v, va_v, kb_v, vb_v, SEG2_BLKS)
    pltpu.sync_copy(rk.at[pl.ds(0, K)], shared_k.at[pl.ds(sid * K, K)])
    pltpu.sync_copy(rv.at[pl.ds(0, K)], shared_v.at[pl.ds(sid * K, K)])
    plsc.subcore_barrier()

    @pl.when(sid == 0)
    def _():
        pltpu.sync_copy(shared_k, ka_v.at[pl.ds(0, NSEG * K)])
        pltpu.sync_copy(shared_v, va_v.at[pl.ds(0, NSEG * K)])
        pltpu.sync_copy(sh_hbm, sh_v)
        fk, fv = _tournament(ka_v, va_v, kb_v, vb_v, NSEG)
        sh = sh_v[...]
        for c in range(4):
            ks_v[pl.ds(16 * c, 16)] = _rev(fk[pl.ds(16 * (3 - c), 16)])
            iv = _rev(fv[pl.ds(16 * (3 - c), 16)]) + sh
            iv = jnp.minimum(jnp.maximum(iv, 0), N - 1)
            idx_v[pl.ds(16 * c, 16)] = iv
        pltpu.async_copy(mem_hbm.at[idx_v], rows_v, sem).wait()
        pltpu.sync_copy(rows_v, ret_hbm)
        pltpu.sync_copy(ks_v, ts_hbm)


def _fused_topk(scores, mem, shift):
    mesh = plsc.VectorSubcoreMesh(core_axis_name="c", subcore_axis_name="s",
                                  num_cores=2, num_subcores=16)
    f = functools.partial(
        pl.kernel,
        out_type=[jax.ShapeDtypeStruct((K, D), jnp.float32),
                  jax.ShapeDtypeStruct((K,), jnp.float32)],
        mesh=mesh,
        compiler_params=pltpu.CompilerParams(needs_layout_passes=False),
        scratch_types=[pltpu.VMEM((SEG2,), jnp.float32),
                       pltpu.VMEM((SEG2,), jnp.float32),
                       pltpu.VMEM((SEG2,), jnp.int32),
                       pltpu.VMEM((SEG2,), jnp.float32),
                       pltpu.VMEM((SEG2,), jnp.int32),
                       pltpu.VMEM((16,), jnp.int32),
                       pltpu.VMEM((K,), jnp.int32),
                       pltpu.VMEM((K, D), jnp.float32),
                       pltpu.VMEM((K,), jnp.float32),
                       pltpu.VMEM_SHARED((NSEG * K,), jnp.float32),
                       pltpu.VMEM_SHARED((NSEG * K,), jnp.int32),
                       pltpu.SemaphoreType.DMA],
    )(_fused_body)
    return f(scores, mem, shift)


def kernel(query, memory_features, k):
    q2 = query.reshape(1, D).astype(jnp.float32)
    scores = _scores(q2, memory_features)
    shift = jnp.broadcast_to(jnp.asarray(k, jnp.int32) - K, (16,))
    retrieved, top_scores = _fused_topk(scores, memory_features, shift)
    return retrieved, top_scores


# speedup vs baseline: 1.1562x; 1.0023x over previous
import functools

import jax
import jax.numpy as jnp
from jax import lax
from jax.experimental import pallas as pl
from jax.experimental.pallas import tpu as pltpu
from jax.experimental.pallas import tpu_sc as plsc

N = 100000
D = 128
K = 64
BLK = 14336
NBLK = 7
NPAD = NBLK * BLK
NEG = float("-inf")


def _scores_body(q_ref, m_ref, o_ref):
    i = pl.program_id(0)
    q = q_ref[...]
    qn = q / jnp.maximum(jnp.sqrt(jnp.sum(q * q)), 1e-12)
    m = m_ref[...]
    ss = jnp.sum(m * m, axis=1, keepdims=True)
    mn = m / jnp.maximum(jnp.sqrt(ss), 1e-12)
    sc = lax.dot_general(qn.astype(jnp.bfloat16), mn.astype(jnp.bfloat16),
                         (((1,), (1,)), ((), ())),
                         preferred_element_type=jnp.float32)
    col = lax.broadcasted_iota(jnp.int32, (1, BLK), 1) + i * BLK
    sc = jnp.where(col < N, sc, NEG)
    o_ref[...] = sc.reshape((BLK,))


def _scores(q2, mem):
    return pl.pallas_call(
        _scores_body,
        grid=(NBLK,),
        in_specs=[
            pl.BlockSpec((1, D), lambda i: (0, 0)),
            pl.BlockSpec((BLK, D), lambda i: (i, 0)),
        ],
        out_specs=pl.BlockSpec((BLK,), lambda i: (i,)),
        out_shape=jax.ShapeDtypeStruct((NPAD,), jnp.float32),
    )(q2, mem)


def _rev(x):
    return lax.rev(x, (0,))


def _srt(k, v):
    return plsc.sort_key_val(k, v)


def _cmpx(ka, va, kb, vb):
    m = ka <= kb
    return (jnp.where(m, ka, kb), jnp.where(m, va, vb),
            jnp.where(m, kb, ka), jnp.where(m, vb, va))


def _merge16(ak, av, bk, bv):
    bk, bv = _rev(bk), _rev(bv)
    lok, lov, hik, hiv = _cmpx(ak, av, bk, bv)
    lok, lov = _srt(lok, lov)
    hik, hiv = _srt(hik, hiv)
    return lok, lov, hik, hiv


def _bitonic64(ks, vs):
    k0, k1, k2, k3 = ks
    v0, v1, v2, v3 = vs
    k0, v0, k2, v2 = _cmpx(k0, v0, k2, v2)
    k1, v1, k3, v3 = _cmpx(k1, v1, k3, v3)
    k0, v0, k1, v1 = _cmpx(k0, v0, k1, v1)
    k2, v2, k3, v3 = _cmpx(k2, v2, k3, v3)
    k0, v0 = _srt(k0, v0)
    k1, v1 = _srt(k1, v1)
    k2, v2 = _srt(k2, v2)
    k3, v3 = _srt(k3, v3)
    return [k0, k1, k2, k3], [v0, v1, v2, v3]


def _sort64(ks, vs):
    k0, v0 = _srt(ks[0], vs[0])
    k1, v1 = _srt(ks[1], vs[1])
    k2, v2 = _srt(ks[2], vs[2])
    k3, v3 = _srt(ks[3], vs[3])
    k0, v0, k1, v1 = _merge16(k0, v0, k1, v1)
    k2, v2, k3, v3 = _merge16(k2, v2, k3, v3)
    return _bitonic64([k0, k1, _rev(k3), _rev(k2)],
                      [v0, v1, _rev(v3), _rev(v2)])


def _topk_merge(rk, rv, bk, bv):
    dk = [_rev(bk[3]), _rev(bk[2]), _rev(bk[1]), _rev(bk[0])]
    dv = [_rev(bv[3]), _rev(bv[2]), _rev(bv[1]), _rev(bv[0])]
    ck, cv = [], []
    for c in range(4):
        m = rk[c] >= dk[c]
        ck.append(jnp.where(m, rk[c], dk[c]))
        cv.append(jnp.where(m, rv[c], dv[c]))
    return _bitonic64(ck, cv)


def _tournament(src_k, src_v, dst_k, dst_v, nblk):
    n = nblk
    while n > 1:
        half = n // 2

        @plsc.parallel_loop(0, half, 1, unroll=2)
        def _m(i, _sk=src_k, _sv=src_v, _dk=dst_k, _dv=dst_v):
            o0 = (2 * i) * K
            o1 = (2 * i + 1) * K
            od = i * K
            ak = [_sk[pl.ds(o0 + 16 * c, 16)] for c in range(4)]
            av = [_sv[pl.ds(o0 + 16 * c, 16)] for c in range(4)]
            bk = [_sk[pl.ds(o1 + 16 * c, 16)] for c in range(4)]
            bv = [_sv[pl.ds(o1 + 16 * c, 16)] for c in range(4)]
            nk, nv = _topk_merge(ak, av, bk, bv)
            for c in range(4):
                _dk[pl.ds(od + 16 * c, 16)] = nk[c]
                _dv[pl.ds(od + 16 * c, 16)] = nv[c]

        if n % 2 == 1:
            ol = (n - 1) * K
            od = half * K
            for c in range(4):
                dst_k[pl.ds(od + 16 * c, 16)] = src_k[pl.ds(ol + 16 * c, 16)]
                dst_v[pl.ds(od + 16 * c, 16)] = src_v[pl.ds(ol + 16 * c, 16)]
        n = half + (n % 2)
        src_k, src_v, dst_k, dst_v = dst_k, dst_v, src_k, src_v
    return src_k, src_v


NPAD2 = 100352
NSEG = 16
SEG2 = NPAD2 // NSEG
SEG2_BLKS = SEG2 // K


def _fused_body(scores_hbm, mem_hbm, sh_hbm, ret_hbm, ts_hbm,
                seg_v, ka_v, va_v, kb_v, vb_v, sh_v, idx_v, rows_v, ks_v,
                shared_k, shared_v, sem):
    sid = lax.axis_index("s")
    base = sid * SEG2
    pltpu.sync_copy(scores_hbm.at[pl.ds(base, SEG2)], seg_v)
    iota = lax.iota(jnp.int32, 16)

    @plsc.parallel_loop(0, SEG2_BLKS, 1, unroll=2)
    def _p1(j):
        off = j * K
        ks = [seg_v[pl.ds(off + 16 * c, 16)] for c in range(4)]
        vs = [iota + (base + off + 16 * c) for c in range(4)]
        sk, sv = _sort64(ks, vs)
        for c in range(4):
            ka_v[pl.ds(off + 16 * c, 16)] = sk[c]
            va_v[pl.ds(off + 16 * c, 16)] = sv[c]

    rk, rv = _tournament(ka_v, va_v, kb_v, vb_v, SEG2_BLKS)
    pltpu.sync_copy(rk.at[pl.ds(0, K)], shared_k.at[pl.ds(sid * K, K)])
    pltpu.sync_copy(rv.at[pl.ds(0, K)], shared_v.at[pl.ds(sid * K, K)])
    plsc.subcore_barrier()

    @pl.when(sid == 0)
    def _():
        pltpu.sync_copy(shared_k, ka_v.at[pl.ds(0, NSEG * K)])
        pltpu.sync_copy(shared_v, va_v.at[pl.ds(0, NSEG * K)])
        pltpu.sync_copy(sh_hbm, sh_v)
        fk, fv = _tournament(ka_v, va_v, kb_v, vb_v, NSEG)
        sh = sh_v[...]
        for c in range(4):
            ks_v[pl.ds(16 * c, 16)] = _rev(fk[pl.ds(16 * (3 - c), 16)])
            iv = _rev(fv[pl.ds(16 * (3 - c), 16)]) + sh
            iv = jnp.minimum(jnp.maximum(iv, 0), N - 1)
            idx_v[pl.ds(16 * c, 16)] = iv
        pltpu.async_copy(mem_hbm.at[idx_v], rows_v, sem).wait()
        pltpu.sync_copy(rows_v, ret_hbm)
        pltpu.sync_copy(ks_v, ts_hbm)


def _fused_topk(scores, mem, shift):
    mesh = plsc.VectorSubcoreMesh(core_axis_name="c", subcore_axis_name="s",
                                  num_cores=2, num_subcores=16)
    f = functools.partial(
        pl.kernel,
        out_type=[jax.ShapeDtypeStruct((K, D), jnp.float32),
                  jax.ShapeDtypeStruct((K,), jnp.float32)],
        mesh=mesh,
        compiler_params=pltpu.CompilerParams(needs_layout_passes=False),
        scratch_types=[pltpu.VMEM((SEG2,), jnp.float32),
                       pltpu.VMEM((SEG2,), jnp.float32),
                       pltpu.VMEM((SEG2,), jnp.int32),
                       pltpu.VMEM((SEG2,), jnp.float32),
                       pltpu.VMEM((SEG2,), jnp.int32),
                       pltpu.VMEM((16,), jnp.int32),
                       pltpu.VMEM((K,), jnp.int32),
                       pltpu.VMEM((K, D), jnp.float32),
                       pltpu.VMEM((K,), jnp.float32),
                       pltpu.VMEM_SHARED((NSEG * K,), jnp.float32),
                       pltpu.VMEM_SHARED((NSEG * K,), jnp.int32),
                       pltpu.SemaphoreType.DMA],
    )(_fused_body)
    return f(scores, mem, shift)


def kernel(query, memory_features, k):
    q2 = query.reshape(1, D).astype(jnp.float32)
    scores = _scores(q2, memory_features)
    shift = jnp.broadcast_to(jnp.asarray(k, jnp.int32) - K, (16,))
    retrieved, top_scores = _fused_topk(scores, memory_features, shift)
    return retrieved, top_scores
